# baseline (speedup 1.0000x reference)
import functools

import jax
import jax.numpy as jnp
from jax import lax
from jax.experimental import pallas as pl
from jax.experimental.pallas import tpu as pltpu
from jax.experimental.pallas import tpu_sc as plsc

D_MODEL = 1024
B_TOTAL = 4 * 8192

_INFO = plsc.get_sparse_core_info()
_NC = _INFO.num_cores
_NS = _INFO.num_subcores
_NW = _NC * _NS
_B_PER_W = B_TOTAL // _NW
_CHUNK = 32
_NCHUNK = _B_PER_W // _CHUNK


def _make_sc_lookup():
    mesh = plsc.VectorSubcoreMesh(core_axis_name="c", subcore_axis_name="s")

    @functools.partial(
        pl.kernel,
        mesh=mesh,
        out_type=jax.ShapeDtypeStruct((B_TOTAL, D_MODEL), jnp.float32),
        scratch_types=[
            pltpu.VMEM((_B_PER_W,), jnp.int32),
            pltpu.VMEM((_CHUNK, D_MODEL), jnp.float32),
            pltpu.SemaphoreType.DMA,
        ],
    )
    def sc_lookup(table_hbm, idx_hbm, out_hbm, idx_v, rows_v, sem):
        wid = lax.axis_index("s") * _NC + lax.axis_index("c")
        base = wid * _B_PER_W
        pltpu.sync_copy(idx_hbm.at[pl.ds(base, _B_PER_W)], idx_v)

        def body(c, _):
            off = c * _CHUNK
            pltpu.async_copy(
                table_hbm.at[idx_v.at[pl.ds(off, _CHUNK)]], rows_v, sem
            ).wait()
            pltpu.sync_copy(rows_v, out_hbm.at[pl.ds(base + off, _CHUNK)])
            return _

        lax.fori_loop(0, _NCHUNK, body, None)

    return sc_lookup


_sc_lookup = _make_sc_lookup()


@jax.jit
def kernel(x, table):
    flat_idx = x.reshape(B_TOTAL).astype(jnp.int32)
    out = _sc_lookup(table, flat_idx)
    return out.reshape(x.shape[0], x.shape[1], D_MODEL)


# speedup vs baseline: 1.0010x; 1.0010x over previous
import functools

import jax
import jax.numpy as jnp
from jax import lax
from jax.experimental import pallas as pl
from jax.experimental.pallas import tpu as pltpu
from jax.experimental.pallas import tpu_sc as plsc

D_MODEL = 1024
B_TOTAL = 4 * 8192

_INFO = plsc.get_sparse_core_info()
_NC = _INFO.num_cores
_NS = _INFO.num_subcores
_NW = _NC * _NS
_B_PER_W = B_TOTAL // _NW
_CHUNK = 32
_NCHUNK = _B_PER_W // _CHUNK


def _make_sc_lookup():
    mesh = plsc.VectorSubcoreMesh(core_axis_name="c", subcore_axis_name="s")

    @functools.partial(
        pl.kernel,
        mesh=mesh,
        out_type=jax.ShapeDtypeStruct((B_TOTAL, D_MODEL), jnp.float32),
        scratch_types=[
            pltpu.VMEM((_B_PER_W,), jnp.int32),
            pltpu.VMEM((_CHUNK, D_MODEL), jnp.float32),
            pltpu.VMEM((_CHUNK, D_MODEL), jnp.float32),
            pltpu.SemaphoreType.DMA,
            pltpu.SemaphoreType.DMA,
            pltpu.SemaphoreType.DMA,
            pltpu.SemaphoreType.DMA,
        ],
    )
    def sc_lookup(table_hbm, idx_hbm, out_hbm, idx_v, rows_a, rows_b,
                  gsem_a, gsem_b, ssem_a, ssem_b):
        wid = lax.axis_index("s") * _NC + lax.axis_index("c")
        base = wid * _B_PER_W
        pltpu.sync_copy(idx_hbm.at[pl.ds(base, _B_PER_W)], idx_v)

        def gather_start(c, buf, sem):
            pltpu.async_copy(
                table_hbm.at[idx_v.at[pl.ds(c * _CHUNK, _CHUNK)]], buf, sem
            )

        def gather_wait(buf, sem):
            pltpu.make_async_copy(
                out_hbm.at[pl.ds(0, _CHUNK)], buf, sem
            ).wait()

        def scatter_start(c, buf, sem):
            pltpu.async_copy(
                buf, out_hbm.at[pl.ds(base + c * _CHUNK, _CHUNK)], sem
            )

        def scatter_wait(buf, sem):
            pltpu.make_async_copy(
                buf, out_hbm.at[pl.ds(base, _CHUNK)], sem
            ).wait()

        gather_start(0, rows_a, gsem_a)

        @pl.loop(0, _NCHUNK, step=2)
        def _(i):
            gather_wait(rows_a, gsem_a)

            @pl.when(i > 0)
            def _():
                scatter_wait(rows_b, ssem_b)

            gather_start(i + 1, rows_b, gsem_b)
            scatter_start(i, rows_a, ssem_a)

            gather_wait(rows_b, gsem_b)
            scatter_wait(rows_a, ssem_a)

            @pl.when(i + 2 < _NCHUNK)
            def _():
                gather_start(i + 2, rows_a, gsem_a)

            scatter_start(i + 1, rows_b, ssem_b)

        scatter_wait(rows_b, ssem_b)

    return sc_lookup


_sc_lookup = _make_sc_lookup()


@jax.jit
def kernel(x, table):
    flat_idx = x.reshape(B_TOTAL).astype(jnp.int32)
    out = _sc_lookup(table, flat_idx)
    return out.reshape(x.shape[0], x.shape[1], D_MODEL)
